# Initial kernel scaffold; baseline (speedup 1.0000x reference)
import jax
import jax.numpy as jnp
from jax.experimental import pallas as pl


def kernel(x, edge_index, batch, W1_rel, b1_rel, W1_root, W2_rel, b2_rel, W2_root):
    raise NotImplementedError("write your pallas kernel here")


# speedup vs baseline: 8.4810x; 8.4810x over previous
import functools

import jax
import jax.numpy as jnp
from jax import lax
from jax.experimental import pallas as pl
from jax.experimental.pallas import tpu as pltpu
from jax.experimental.pallas import tpu_sc as plsc

N = 10000
E = 320000
D = 128
HD = 64
G = 64

NPAD = 10240
CHUNK = 128
NROWS = 2560
E_PAD = NROWS * CHUNK
NCH = NROWS // 16
ROWS_PT = NPAD // 16


def _sc_segment_sum(xs, srcb, dst2d, zeros_rows):
    @functools.partial(
        pl.kernel,
        out_type=jax.ShapeDtypeStruct((2 * NPAD, HD), jnp.float32),
        mesh=plsc.VectorSubcoreMesh(core_axis_name="c", subcore_axis_name="s"),
        compiler_params=pltpu.CompilerParams(use_tc_tiling_on_sc=False),
        scratch_types=[
            pltpu.VMEM((NCH, CHUNK), jnp.int32),
            pltpu.VMEM((NCH, CHUNK), jnp.int32),
            pltpu.VMEM((CHUNK, HD), jnp.float32),
            pltpu.VMEM((CHUNK, HD), jnp.float32),
            pltpu.VMEM_SHARED((NPAD, HD), jnp.float32),
            pltpu.SemaphoreType.DMA,
            pltpu.SemaphoreType.DMA,
        ],
    )
    def k(x_h, src_h, dst_h, z_h, out_h, src_v, dst_v, rowa, rowb, acc, sema, semb):
        c = lax.axis_index("c")
        s = lax.axis_index("s")

        pltpu.sync_copy(src_h.at[pl.ds(c * NROWS + s * NCH, NCH)], src_v)
        pltpu.sync_copy(dst_h.at[pl.ds(s * NCH, NCH)], dst_v)
        pltpu.sync_copy(z_h, acc.at[pl.ds(s * ROWS_PT, ROWS_PT)])
        plsc.subcore_barrier()

        pltpu.async_copy(x_h.at[src_v.at[0]], rowa, sema)

        def body(i, _):
            j = i * 2
            pltpu.async_copy(x_h.at[src_v.at[j + 1]], rowb, semb)
            pltpu.make_async_copy(x_h.at[src_v.at[j]], rowa, sema).wait()
            pltpu.sync_copy(rowa, acc.at[dst_v.at[j]], add=True)

            @pl.when(j + 2 < NCH)
            def _():
                pltpu.async_copy(x_h.at[src_v.at[j + 2]], rowa, sema)

            pltpu.make_async_copy(x_h.at[src_v.at[j + 1]], rowb, semb).wait()
            pltpu.sync_copy(rowb, acc.at[dst_v.at[j + 1]], add=True)
            return 0

        lax.fori_loop(0, NCH // 2, body, 0)
        plsc.subcore_barrier()
        pltpu.sync_copy(
            acc.at[pl.ds(s * ROWS_PT, ROWS_PT)],
            out_h.at[pl.ds(c * NPAD + s * ROWS_PT, ROWS_PT)],
        )

    return k(xs, srcb, dst2d, zeros_rows)


def _split(a):
    return jnp.concatenate([a[:, :HD], a[:, HD:]], axis=0)


def _unsplit(a):
    return jnp.concatenate([a[:NPAD], a[NPAD:]], axis=1)


def _tc_layer(agg, xin, w_rel, w_root, b):
    BN = 1280

    def body(a_r, x_r, wr_r, wt_r, b_r, o_r):
        h = lax.dot_general(a_r[...], wr_r[...], (((1,), (1,)), ((), ())),
                            preferred_element_type=jnp.float32)
        h = h + lax.dot_general(x_r[...], wt_r[...], (((1,), (1,)), ((), ())),
                                preferred_element_type=jnp.float32)
        o_r[...] = jnp.tanh(h + b_r[...])

    row_spec = pl.BlockSpec((BN, D), lambda i: (i, 0))
    w_spec = pl.BlockSpec((D, D), lambda i: (0, 0))
    return pl.pallas_call(
        body,
        grid=(NPAD // BN,),
        in_specs=[row_spec, row_spec, w_spec, w_spec,
                  pl.BlockSpec((1, D), lambda i: (0, 0))],
        out_specs=row_spec,
        out_shape=jax.ShapeDtypeStruct((NPAD, D), jnp.float32),
    )(agg, xin, w_rel, w_root, b)


def _tc_layer_pool(agg, xin, w_rel, w_root, b, batch3d):
    BN = 128

    def body(a_r, x_r, wr_r, wt_r, b_r, bat_r, o_r, sums, counts):
        i = pl.program_id(0)

        @pl.when(i == 0)
        def _():
            sums[...] = jnp.zeros_like(sums)
            counts[...] = jnp.zeros_like(counts)

        h = lax.dot_general(a_r[...], wr_r[...], (((1,), (1,)), ((), ())),
                            preferred_element_type=jnp.float32)
        h = h + lax.dot_general(x_r[...], wt_r[...], (((1,), (1,)), ((), ())),
                                preferred_element_type=jnp.float32)
        x2 = jnp.tanh(h + b_r[...])

        bat = bat_r[...].reshape(1, BN)
        oh = (lax.broadcasted_iota(jnp.int32, (G, BN), 0)
              == jnp.broadcast_to(bat, (G, BN))).astype(jnp.float32)
        sums[...] += lax.dot_general(oh, x2, (((1,), (0,)), ((), ())),
                                     preferred_element_type=jnp.float32)
        ones = jnp.ones((BN, D), jnp.float32)
        counts[...] += lax.dot_general(oh, ones, (((1,), (0,)), ((), ())),
                                       preferred_element_type=jnp.float32)

        @pl.when(i == pl.num_programs(0) - 1)
        def _():
            o_r[...] = sums[...] / jnp.maximum(counts[...], 1.0)

    row_spec = pl.BlockSpec((BN, D), lambda i: (i, 0))
    w_spec = pl.BlockSpec((D, D), lambda i: (0, 0))
    return pl.pallas_call(
        body,
        grid=(NPAD // BN,),
        in_specs=[row_spec, row_spec, w_spec, w_spec,
                  pl.BlockSpec((1, D), lambda i: (0, 0)),
                  pl.BlockSpec((1, 1, BN), lambda i: (i, 0, 0))],
        out_specs=pl.BlockSpec((G, D), lambda i: (0, 0)),
        out_shape=jax.ShapeDtypeStruct((G, D), jnp.float32),
        scratch_shapes=[pltpu.VMEM((G, D), jnp.float32),
                        pltpu.VMEM((G, D), jnp.float32)],
    )(agg, xin, w_rel, w_root, b, batch3d)


def kernel(x, edge_index, batch, W1_rel, b1_rel, W1_root, W2_rel, b2_rel, W2_root):
    x_pad = jnp.concatenate([x, jnp.zeros((NPAD - N, D), x.dtype)], axis=0)
    pad_idx = N + jnp.arange(E_PAD - E, dtype=jnp.int32) % (NPAD - N)
    src2d = jnp.concatenate([edge_index[0], pad_idx]).reshape(NROWS, CHUNK)
    srcb = jnp.concatenate([src2d, src2d + NPAD], axis=0)
    dst2d = jnp.concatenate([edge_index[1], pad_idx]).reshape(NROWS, CHUNK)
    batch3d = jnp.concatenate(
        [batch, jnp.full((NPAD - N,), G, jnp.int32)]).reshape(NPAD // 128, 1, 128)
    zeros_rows = jnp.zeros((ROWS_PT, HD), jnp.float32)
    b1 = b1_rel.reshape(1, D)
    b2 = b2_rel.reshape(1, D)

    agg1 = _unsplit(_sc_segment_sum(_split(x_pad), srcb, dst2d, zeros_rows))
    x1 = _tc_layer(agg1, x_pad, W1_rel, W1_root, b1)
    agg2 = _unsplit(_sc_segment_sum(_split(x1), srcb, dst2d, zeros_rows))
    return _tc_layer_pool(agg2, x1, W2_rel, W2_root, b2, batch3d)


# speedup vs baseline: 8.5867x; 1.0125x over previous
import functools

import jax
import jax.numpy as jnp
from jax import lax
from jax.experimental import pallas as pl
from jax.experimental.pallas import tpu as pltpu
from jax.experimental.pallas import tpu_sc as plsc

N = 10000
E = 320000
D = 128
HD = 64
G = 64

NPAD = 10240
CHUNK = 128
NROWS = 2560
E_PAD = NROWS * CHUNK
NCH = NROWS // 16
ROWS_PT = NPAD // 16


def _sc_segment_sum(xs, srcb, dst2d, zeros_rows):
    @functools.partial(
        pl.kernel,
        out_type=jax.ShapeDtypeStruct((2 * NPAD, HD), jnp.float32),
        mesh=plsc.VectorSubcoreMesh(core_axis_name="c", subcore_axis_name="s"),
        compiler_params=pltpu.CompilerParams(use_tc_tiling_on_sc=False),
        scratch_types=[
            pltpu.VMEM((NCH, CHUNK), jnp.int32),
            pltpu.VMEM((NCH, CHUNK), jnp.int32),
            pltpu.VMEM((CHUNK, HD), jnp.float32),
            pltpu.VMEM((CHUNK, HD), jnp.float32),
            pltpu.VMEM_SHARED((NPAD, HD), jnp.float32),
            pltpu.SemaphoreType.DMA,
            pltpu.SemaphoreType.DMA,
        ],
    )
    def k(x_h, src_h, dst_h, z_h, out_h, src_v, dst_v, rowa, rowb, acc, sema, semb):
        c = lax.axis_index("c")
        s = lax.axis_index("s")

        pltpu.sync_copy(src_h.at[pl.ds(c * NROWS + s * NCH, NCH)], src_v)
        pltpu.sync_copy(dst_h.at[pl.ds(s * NCH, NCH)], dst_v)
        pltpu.sync_copy(z_h.at[pl.ds(s * ROWS_PT, ROWS_PT)],
                        acc.at[pl.ds(s * ROWS_PT, ROWS_PT)])
        plsc.subcore_barrier()

        pltpu.async_copy(x_h.at[src_v.at[0]], rowa, sema)

        def body(i, _):
            j = i * 2
            pltpu.async_copy(x_h.at[src_v.at[j + 1]], rowb, semb)
            pltpu.make_async_copy(x_h.at[src_v.at[j]], rowa, sema).wait()
            pltpu.sync_copy(rowa, acc.at[dst_v.at[j]], add=True)

            @pl.when(j + 2 < NCH)
            def _():
                pltpu.async_copy(x_h.at[src_v.at[j + 2]], rowa, sema)

            pltpu.make_async_copy(x_h.at[src_v.at[j + 1]], rowb, semb).wait()
            pltpu.sync_copy(rowb, acc.at[dst_v.at[j + 1]], add=True)
            return 0

        lax.fori_loop(0, NCH // 2, body, 0)
        plsc.subcore_barrier()
        pltpu.sync_copy(
            acc.at[pl.ds(s * ROWS_PT, ROWS_PT)],
            out_h.at[pl.ds(c * NPAD + s * ROWS_PT, ROWS_PT)],
        )

    return k(xs, srcb, dst2d, zeros_rows)


def _tc_layer(aggs, xs, w_rel, w_root, b):
    BN = 1280
    NB = NPAD // BN

    def body(al_r, ah_r, xl_r, xh_r, wr_r, wt_r, b_r, o_r):
        f = pl.program_id(0)
        wr = wr_r[...]
        wt = wt_r[...]
        h = lax.dot_general(al_r[...], wr[:, :HD], (((1,), (1,)), ((), ())),
                            preferred_element_type=jnp.float32)
        h = h + lax.dot_general(ah_r[...], wr[:, HD:], (((1,), (1,)), ((), ())),
                                preferred_element_type=jnp.float32)
        h = h + lax.dot_general(xl_r[...], wt[:, :HD], (((1,), (1,)), ((), ())),
                                preferred_element_type=jnp.float32)
        h = h + lax.dot_general(xh_r[...], wt[:, HD:], (((1,), (1,)), ((), ())),
                                preferred_element_type=jnp.float32)
        t = jnp.tanh(h + b_r[...])
        o_r[...] = jnp.where(f == 0, t[:, :HD], t[:, HD:])

    lo = pl.BlockSpec((BN, HD), lambda f, i: (i, 0))
    hi = pl.BlockSpec((BN, HD), lambda f, i: (NB + i, 0))
    w_spec = pl.BlockSpec((D, D), lambda f, i: (0, 0))
    return pl.pallas_call(
        body,
        grid=(2, NB),
        in_specs=[lo, hi, lo, hi, w_spec, w_spec,
                  pl.BlockSpec((1, D), lambda f, i: (0, 0))],
        out_specs=pl.BlockSpec((BN, HD), lambda f, i: (f * NB + i, 0)),
        out_shape=jax.ShapeDtypeStruct((2 * NPAD, HD), jnp.float32),
    )(aggs, aggs, xs, xs, w_rel, w_root, b)


def _tc_layer_pool(aggs, xs, w_rel, w_root, b, batch3d):
    BN = 128
    NB = NPAD // BN

    def body(al_r, ah_r, xl_r, xh_r, wr_r, wt_r, b_r, bat_r, o_r, sums, counts):
        i = pl.program_id(0)

        @pl.when(i == 0)
        def _():
            sums[...] = jnp.zeros_like(sums)
            counts[...] = jnp.zeros_like(counts)

        wr = wr_r[...]
        wt = wt_r[...]
        h = lax.dot_general(al_r[...], wr[:, :HD], (((1,), (1,)), ((), ())),
                            preferred_element_type=jnp.float32)
        h = h + lax.dot_general(ah_r[...], wr[:, HD:], (((1,), (1,)), ((), ())),
                                preferred_element_type=jnp.float32)
        h = h + lax.dot_general(xl_r[...], wt[:, :HD], (((1,), (1,)), ((), ())),
                                preferred_element_type=jnp.float32)
        h = h + lax.dot_general(xh_r[...], wt[:, HD:], (((1,), (1,)), ((), ())),
                                preferred_element_type=jnp.float32)
        x2 = jnp.tanh(h + b_r[...])

        bat = bat_r[...].reshape(1, BN)
        oh = (lax.broadcasted_iota(jnp.int32, (G, BN), 0)
              == jnp.broadcast_to(bat, (G, BN))).astype(jnp.float32)
        sums[...] += lax.dot_general(oh, x2, (((1,), (0,)), ((), ())),
                                     preferred_element_type=jnp.float32)
        ones = jnp.ones((BN, D), jnp.float32)
        counts[...] += lax.dot_general(oh, ones, (((1,), (0,)), ((), ())),
                                       preferred_element_type=jnp.float32)

        @pl.when(i == pl.num_programs(0) - 1)
        def _():
            o_r[...] = sums[...] / jnp.maximum(counts[...], 1.0)

    lo = pl.BlockSpec((BN, HD), lambda i: (i, 0))
    hi = pl.BlockSpec((BN, HD), lambda i: (NB + i, 0))
    w_spec = pl.BlockSpec((D, D), lambda i: (0, 0))
    return pl.pallas_call(
        body,
        grid=(NB,),
        in_specs=[lo, hi, lo, hi, w_spec, w_spec,
                  pl.BlockSpec((1, D), lambda i: (0, 0)),
                  pl.BlockSpec((1, 1, BN), lambda i: (i, 0, 0))],
        out_specs=pl.BlockSpec((G, D), lambda i: (0, 0)),
        out_shape=jax.ShapeDtypeStruct((G, D), jnp.float32),
        scratch_shapes=[pltpu.VMEM((G, D), jnp.float32),
                        pltpu.VMEM((G, D), jnp.float32)],
    )(aggs, aggs, xs, xs, w_rel, w_root, b, batch3d)


def kernel(x, edge_index, batch, W1_rel, b1_rel, W1_root, W2_rel, b2_rel, W2_root):
    zpad = jnp.zeros((NPAD - N, HD), x.dtype)
    xs = jnp.concatenate([x[:, :HD], zpad, x[:, HD:], zpad], axis=0)
    pad_idx = N + jnp.arange(E_PAD - E, dtype=jnp.int32) % (NPAD - N)
    src2d = jnp.concatenate([edge_index[0], pad_idx]).reshape(NROWS, CHUNK)
    srcb = jnp.concatenate([src2d, src2d + NPAD], axis=0)
    dst2d = jnp.concatenate([edge_index[1], pad_idx]).reshape(NROWS, CHUNK)
    batch3d = jnp.concatenate(
        [batch, jnp.full((NPAD - N,), G, jnp.int32)]).reshape(NPAD // 128, 1, 128)
    zeros_rows = jnp.zeros((NPAD, HD), jnp.float32)
    b1 = b1_rel.reshape(1, D)
    b2 = b2_rel.reshape(1, D)

    agg1 = _sc_segment_sum(xs, srcb, dst2d, zeros_rows)
    x1 = _tc_layer(agg1, xs, W1_rel, W1_root, b1)
    agg2 = _sc_segment_sum(x1, srcb, dst2d, zeros_rows)
    return _tc_layer_pool(agg2, x1, W2_rel, W2_root, b2, batch3d)
